# baseline (speedup 1.0000x reference)
import dataclasses
import functools

import numpy as np
import jax
import jax.numpy as jnp
from jax import lax
from jax.experimental import pallas as pl
from jax.experimental.pallas import tpu as pltpu
from jax.experimental.pallas import tpu_sc as plsc

_BINS = 10
_EPS = 1e-8
_NSUB = 16
_ACCR = 8


def _edges_f32():
    e = np.arange(_BINS + 1, dtype=np.float32) / np.float32(_BINS)
    e[-1] = np.float32(e[-1] + np.float32(1e-6))
    return [float(v) for v in e]


def _row_sum_via_transpose(a):
    r, c = a.shape
    nfull = c // 128
    acc = a[:, :128]
    for i in range(1, nfull):
        acc = acc + a[:, i * 128:(i + 1) * 128]
    tail = c - nfull * 128
    if tail:
        acc = acc + jnp.pad(a[:, nfull * 128:], ((0, 0), (0, 128 - tail)))
    return jnp.sum(acc.T, axis=0)


def _tc_dense_body(tgt_ref, pred_ref, g_ref, loss_ref):
    x = pred_ref[...]
    r, c = x.shape
    t = tgt_ref[0, 0, :]
    col = lax.broadcasted_iota(jnp.int32, (r, c), 1)
    e = jnp.exp(x)
    em = jnp.where(col == t[:, None], e, 0.0)
    s = _row_sum_via_transpose(e)
    em_s = _row_sum_via_transpose(em)
    m = jnp.log(em_s)
    ls = jnp.log(s + _EPS)
    g_ref[...] = 1.0 - em_s / s
    loss_ref[...] = ls - m


def _tc_dense(pred, target, row_start, nrows, row_block=2048):
    bsz, csz = pred.shape
    nb = nrows // row_block
    b0 = row_start // row_block
    tgt3 = target.reshape(bsz // row_block, 1, row_block)
    g1, loss1 = pl.pallas_call(
        _tc_dense_body,
        grid=(nb,),
        in_specs=[
            pl.BlockSpec((1, 1, row_block), lambda i: (i + b0, 0, 0)),
            pl.BlockSpec((row_block, csz), lambda i: (i + b0, 0)),
        ],
        out_specs=[
            pl.BlockSpec((row_block,), lambda i: (i,)),
            pl.BlockSpec((row_block,), lambda i: (i,)),
        ],
        out_shape=[
            jax.ShapeDtypeStruct((nrows,), jnp.float32),
            jax.ShapeDtypeStruct((nrows,), jnp.float32),
        ],
    )(tgt3, pred)
    return g1, loss1


def _sc_compiler_params():
    cp = pltpu.CompilerParams()
    if "needs_layout_passes" in pltpu.CompilerParams.__dataclass_fields__:
        cp = dataclasses.replace(cp, needs_layout_passes=False)
    return cp


def _sc_hist_body(pw, g_hbm, loss_hbm, parts_hbm, g_v, l_v, acc_v, sem):
    cid = lax.axis_index("c")
    sid = lax.axis_index("s")
    nv = pw // 16

    @pl.when(cid == 0)
    def _work():
        base = sid * pw
        cp_g = pltpu.async_copy(g_hbm.at[pl.ds(base, pw)], g_v, sem)
        cp_l = pltpu.async_copy(loss_hbm.at[pl.ds(base, pw)], l_v, sem)
        cp_g.wait()
        cp_l.wait()

        zero = jnp.zeros((16,), jnp.float32)
        ones = jnp.full((16,), 1.0, jnp.float32)
        for r in range(_ACCR):
            acc_v[r] = zero

        edges = _edges_f32()

        @pl.loop(0, nv)
        def _binloop(k):
            sl = pl.ds(k * 16, 16)
            g = g_v[sl]
            loss = l_v[sl]
            nge = jnp.zeros((16,), jnp.int32)
            for ev in edges:
                nge = nge + jnp.where(g >= ev, 1, 0).astype(jnp.int32)
            bin_idx = jnp.minimum(jnp.maximum(nge - 1, 0), _BINS - 1)
            plsc.addupdate_scatter(acc_v.at[0], [bin_idx], ones)
            plsc.addupdate_scatter(acc_v.at[1], [bin_idx], loss)

        pltpu.sync_copy(acc_v, parts_hbm.at[pl.ds(sid * _ACCR, _ACCR)])


def _sc_hist(g, loss):
    n = g.shape[0]
    pw = n // _NSUB
    mesh = plsc.VectorSubcoreMesh(core_axis_name="c", subcore_axis_name="s")
    k = pl.kernel(
        functools.partial(_sc_hist_body, pw),
        out_type=jax.ShapeDtypeStruct((_NSUB * _ACCR, 16), jnp.float32),
        mesh=mesh,
        scratch_types=[
            pltpu.VMEM((pw,), jnp.float32),
            pltpu.VMEM((pw,), jnp.float32),
            pltpu.VMEM((_ACCR, 16), jnp.float32),
            pltpu.SemaphoreType.DMA,
        ],
        compiler_params=_sc_compiler_params(),
    )
    return k(g, loss)


def _sc_final_body(p0_hbm, p1_hbm, out_hbm, gat_v, out_v, sem):
    cid = lax.axis_index("c")
    sid = lax.axis_index("s")

    @pl.when((cid == 0) & (sid == 0))
    def _final():
        cp0 = pltpu.async_copy(p0_hbm, gat_v.at[pl.ds(0, _NSUB * _ACCR)], sem)
        cp1 = pltpu.async_copy(
            p1_hbm, gat_v.at[pl.ds(_NSUB * _ACCR, _NSUB * _ACCR)], sem)
        cp0.wait()
        cp1.wait()
        cnt16 = gat_v[0]
        l16 = gat_v[1]
        for i in range(1, 2 * _NSUB):
            cnt16 = cnt16 + gat_v[i * _ACCR]
            l16 = l16 + gat_v[i * _ACCR + 1]
        mask = cnt16 > 0.0
        n = jnp.sum(jnp.where(mask, 1.0, 0.0))
        termv = jnp.where(mask, l16 / jnp.maximum(cnt16, 1.0), 0.0)
        tot = jnp.sum(termv)
        res_v = jnp.full((16,), tot, jnp.float32) / jnp.maximum(
            jnp.full((16,), n, jnp.float32), 1.0)
        out_v[...] = res_v
        pltpu.sync_copy(out_v, out_hbm)


def _sc_final(p0, p1):
    mesh = plsc.VectorSubcoreMesh(core_axis_name="c", subcore_axis_name="s")
    k = pl.kernel(
        _sc_final_body,
        out_type=jax.ShapeDtypeStruct((16,), jnp.float32),
        mesh=mesh,
        scratch_types=[
            pltpu.VMEM((2 * _NSUB * _ACCR, 16), jnp.float32),
            pltpu.VMEM((16,), jnp.float32),
            pltpu.SemaphoreType.DMA,
        ],
        compiler_params=_sc_compiler_params(),
    )
    return k(p0, p1)


def kernel(pred, target):
    bsz = pred.shape[0]
    half = bsz // 2
    g0, l0 = _tc_dense(pred, target, 0, half)
    g1, l1 = _tc_dense(pred, target, half, half)
    p0 = _sc_hist(g0, l0)
    p1 = _sc_hist(g1, l1)
    out16 = _sc_final(p0, p1)
    return out16[0]


# speedup vs baseline: 1.0555x; 1.0555x over previous
import dataclasses
import functools

import numpy as np
import jax
import jax.numpy as jnp
from jax import lax
from jax.experimental import pallas as pl
from jax.experimental.pallas import tpu as pltpu
from jax.experimental.pallas import tpu_sc as plsc

_BINS = 10
_EPS = 1e-8
_B = 16384
_C = 1000
_NSUB = 16
_PW = _B // _NSUB
_NV = _PW // 16


def _edges_f32():
    e = np.arange(_BINS + 1, dtype=np.float32) / np.float32(_BINS)
    e[-1] = np.float32(e[-1] + np.float32(1e-6))
    return [float(v) for v in e]


def _row_sum_via_transpose(a):
    r, c = a.shape
    nfull = c // 128
    acc = a[:, :128]
    for i in range(1, nfull):
        acc = acc + a[:, i * 128:(i + 1) * 128]
    tail = c - nfull * 128
    if tail:
        acc = acc + jnp.pad(a[:, nfull * 128:], ((0, 0), (0, 128 - tail)))
    return jnp.sum(acc.T, axis=0)


def _tc_dense_body(tgt_ref, pred_ref, g_ref, loss_ref):
    x = pred_ref[...]
    r, c = x.shape
    t = tgt_ref[0, 0, :]
    col = lax.broadcasted_iota(jnp.int32, (r, c), 1)
    e = jnp.exp(x)
    em = jnp.where(col == t[:, None], e, 0.0)
    s = _row_sum_via_transpose(e)
    em_s = _row_sum_via_transpose(em)
    m = jnp.log(em_s)
    ls = jnp.log(s + _EPS)
    g_ref[...] = 1.0 - em_s / s
    loss_ref[...] = ls - m


def _tc_dense(pred, target, row_block=2048):
    bsz, csz = pred.shape
    nb = bsz // row_block
    tgt3 = target.reshape(nb, 1, row_block)
    g1, loss1 = pl.pallas_call(
        _tc_dense_body,
        grid=(nb,),
        in_specs=[
            pl.BlockSpec((1, 1, row_block), lambda i: (i, 0, 0)),
            pl.BlockSpec((row_block, csz), lambda i: (i, 0)),
        ],
        out_specs=[
            pl.BlockSpec((row_block,), lambda i: (i,)),
            pl.BlockSpec((row_block,), lambda i: (i,)),
        ],
        out_shape=[
            jax.ShapeDtypeStruct((bsz,), jnp.float32),
            jax.ShapeDtypeStruct((bsz,), jnp.float32),
        ],
    )(tgt3, pred)
    return g1, loss1


_ACCR = 8


def _sc_body(g_hbm, loss_hbm, out_hbm, parts_hbm,
             g_v, l_v, acc_v, gat_v, out_v, sem):
    cid = lax.axis_index("c")
    sid = lax.axis_index("s")

    @pl.when(cid == 0)
    def _work():
        base = sid * _PW
        cp_g = pltpu.async_copy(g_hbm.at[pl.ds(base, _PW)], g_v, sem)
        cp_l = pltpu.async_copy(loss_hbm.at[pl.ds(base, _PW)], l_v, sem)
        cp_g.wait()
        cp_l.wait()

        zero = jnp.zeros((16,), jnp.float32)
        ones = jnp.full((16,), 1.0, jnp.float32)
        for r in range(_ACCR):
            acc_v[r] = zero

        edges = _edges_f32()

        @pl.loop(0, _NV)
        def _binloop(k):
            sl = pl.ds(k * 16, 16)
            g = g_v[sl]
            loss = l_v[sl]
            nge = jnp.zeros((16,), jnp.int32)
            for ev in edges:
                nge = nge + jnp.where(g >= ev, 1, 0).astype(jnp.int32)
            bin_idx = jnp.minimum(jnp.maximum(nge - 1, 0), _BINS - 1)
            plsc.addupdate_scatter(acc_v.at[0], [bin_idx], ones)
            plsc.addupdate_scatter(acc_v.at[1], [bin_idx], loss)

        pltpu.sync_copy(acc_v, parts_hbm.at[pl.ds(sid * _ACCR, _ACCR)])

    plsc.subcore_barrier()

    @pl.when((cid == 0) & (sid == 0))
    def _final():
        pltpu.sync_copy(parts_hbm, gat_v)
        cnt16 = gat_v[0]
        l16 = gat_v[1]
        for i in range(1, _NSUB):
            cnt16 = cnt16 + gat_v[i * _ACCR]
            l16 = l16 + gat_v[i * _ACCR + 1]
        mask = cnt16 > 0.0
        n = jnp.sum(jnp.where(mask, 1.0, 0.0))
        termv = jnp.where(mask, l16 / jnp.maximum(cnt16, 1.0), 0.0)
        tot = jnp.sum(termv)
        res_v = jnp.full((16,), tot, jnp.float32) / jnp.maximum(
            jnp.full((16,), n, jnp.float32), 1.0)
        out_v[...] = res_v
        pltpu.sync_copy(out_v, out_hbm)


def _sc_stage(g, loss):
    mesh = plsc.VectorSubcoreMesh(core_axis_name="c", subcore_axis_name="s")
    cp = pltpu.CompilerParams()
    if "needs_layout_passes" in pltpu.CompilerParams.__dataclass_fields__:
        cp = dataclasses.replace(cp, needs_layout_passes=False)
    k = pl.kernel(
        _sc_body,
        out_type=[
            jax.ShapeDtypeStruct((16,), jnp.float32),
            jax.ShapeDtypeStruct((_NSUB * _ACCR, 16), jnp.float32),
        ],
        mesh=mesh,
        scratch_types=[
            pltpu.VMEM((_PW,), jnp.float32),
            pltpu.VMEM((_PW,), jnp.float32),
            pltpu.VMEM((_ACCR, 16), jnp.float32),
            pltpu.VMEM((_NSUB * _ACCR, 16), jnp.float32),
            pltpu.VMEM((16,), jnp.float32),
            pltpu.SemaphoreType.DMA,
        ],
        compiler_params=cp,
    )
    out16, _parts = k(g, loss)
    return out16


def kernel(pred, target):
    g, loss = _tc_dense(pred, target)
    out16 = _sc_stage(g, loss)
    return out16[0]
